# Initial kernel scaffold; baseline (speedup 1.0000x reference)
import jax
import jax.numpy as jnp
from jax.experimental import pallas as pl


def kernel(M, nodes):
    raise NotImplementedError("write your pallas kernel here")


# speedup vs baseline: 4.0842x; 4.0842x over previous
import jax
import jax.numpy as jnp
from jax import lax
from jax.experimental import pallas as pl
from jax.experimental.pallas import tpu as pltpu
from jax.experimental.pallas import tpu_sc as plsc

N = 10000
D = 128
NUM_MSG = 10000
NC = 2
NS = 16
L = 16

PASS = 320
CK = 80
NH = N // NC
NHP = 5120
DUMMY = NH
RT = 320


def _body(m_hbm, nodes_hbm, out_hbm,
          sums_sh, cnts_sh,
          rows_v, idx_v, lidx_v, riota_v, ones_v, z16_v):
    core = lax.axis_index("c")
    sub = lax.axis_index("s")
    zvec = jnp.zeros((L,), jnp.float32)
    onevec = jnp.ones((L,), jnp.float32)
    nbase = core * NH
    rbase = sub * RT

    def _init(i, _):
        for c in range(D // L):
            rows_v[i, pl.ds(c * L, L)] = zvec
        ones_v[i, :] = onevec
        z16_v[i, :] = zvec
        return 0
    lax.fori_loop(0, PASS, _init, 0)
    for g in range(RT // L):
        riota_v[g * L // CK, pl.ds(g * L % CK, L)] = \
            lax.iota(jnp.int32, L) + (rbase + g * L)

    pltpu.sync_copy(rows_v, sums_sh.at[pl.ds(rbase, RT)])
    for j in range(RT // CK):
        pltpu.sync_copy(z16_v.at[pl.ds(j * CK, CK)], cnts_sh.at[riota_v.at[j]])
    plsc.subcore_barrier()

    def _pass(mbase, ngroups):
        cnt = ngroups * L
        pltpu.sync_copy(nodes_hbm.at[pl.ds(mbase, cnt)], idx_v.at[pl.ds(0, cnt)])
        pltpu.sync_copy(m_hbm.at[pl.ds(mbase, cnt)], rows_v.at[pl.ds(0, cnt)])
        for g in range(ngroups):
            v = idx_v[pl.ds(g * L, L)]
            inr = (v >= nbase) & (v < nbase + NH)
            lidx_v[g * L // CK, pl.ds(g * L % CK, L)] = \
                jnp.where(inr, v - nbase, DUMMY)
        for j in range(cnt // CK):
            pltpu.sync_copy(rows_v.at[pl.ds(j * CK, CK)],
                            sums_sh.at[lidx_v.at[j]], add=True)
            pltpu.sync_copy(ones_v.at[pl.ds(j * CK, CK)],
                            cnts_sh.at[lidx_v.at[j]], add=True)

    _pass(sub * 640, PASS // L)

    @pl.when(sub < NS - 1)
    def _full_second_pass():
        _pass(sub * 640 + PASS, PASS // L)

    @pl.when(sub == NS - 1)
    def _short_second_pass():
        _pass((NS - 1) * 640 + PASS, (NUM_MSG - (NS - 1) * 640 - PASS) // L)

    plsc.subcore_barrier()

    pltpu.sync_copy(sums_sh.at[pl.ds(rbase, RT)], rows_v)
    for j in range(RT // CK):
        pltpu.sync_copy(cnts_sh.at[riota_v.at[j]], z16_v.at[pl.ds(j * CK, CK)])

    nrows = jnp.where(sub == NS - 1, NH - (NS - 1) * RT, RT)

    def _scale(r, _):
        cvec = z16_v[r, :]
        s = jnp.where(cvec > 0, 1.0 / cvec, 0.0)[0]
        for c in range(D // L):
            rows_v[r, pl.ds(c * L, L)] = rows_v[r, pl.ds(c * L, L)] * s
        return 0
    lax.fori_loop(0, nrows, _scale, 0)

    @pl.when(sub < NS - 1)
    def _write_full():
        pltpu.sync_copy(rows_v, out_hbm.at[pl.ds(core * NH + rbase, RT)])

    @pl.when(sub == NS - 1)
    def _write_short():
        last = NH - (NS - 1) * RT
        pltpu.sync_copy(rows_v.at[pl.ds(0, last)],
                        out_hbm.at[pl.ds(core * NH + (NS - 1) * RT, last)])


_agg = pl.kernel(
    _body,
    out_type=jax.ShapeDtypeStruct((N, D), jnp.float32),
    mesh=plsc.VectorSubcoreMesh(core_axis_name="c", subcore_axis_name="s",
                                num_cores=NC, num_subcores=NS),
    compiler_params=pltpu.CompilerParams(use_tc_tiling_on_sc=False),
    scratch_types=[
        pltpu.VMEM_SHARED((NHP, D), jnp.float32),
        pltpu.VMEM_SHARED((NHP, L), jnp.float32),
        pltpu.VMEM((PASS, D), jnp.float32),
        pltpu.VMEM((PASS,), jnp.int32),
        pltpu.VMEM((PASS // CK, CK), jnp.int32),
        pltpu.VMEM((RT // CK, CK), jnp.int32),
        pltpu.VMEM((PASS, L), jnp.float32),
        pltpu.VMEM((PASS, L), jnp.float32),
    ],
)


@jax.jit
def kernel(M, nodes):
    return _agg(M, nodes.astype(jnp.int32))


# speedup vs baseline: 4.3151x; 1.0565x over previous
import jax
import jax.numpy as jnp
from jax import lax
from jax.experimental import pallas as pl
from jax.experimental.pallas import tpu as pltpu
from jax.experimental.pallas import tpu_sc as plsc

N = 10000
D = 128
NUM_MSG = 10000
NC = 2
NS = 16
L = 16

HB = 160
CK = 80
MPT = 640
NH = N // NC
NHP = 5120
DUMMY = NH
RT = 320


def _body(m_hbm, nodes_hbm, zeros_hbm, out_hbm,
          sums_sh, cnts_sh,
          rows2, idx2, lidx2, riota_v, ones_v, z16_v,
          sem_ldn, sem_ldr, sem_scr, sem_sco, sem_wr):
    core = lax.axis_index("c")
    sub = lax.axis_index("s")
    zvec = jnp.zeros((L,), jnp.float32)
    onevec = jnp.ones((L,), jnp.float32)
    nbase = core * NH
    rbase = sub * RT
    mbase = sub * MPT

    def fire_load(step, buf, rows):
        mb = mbase + step * HB
        dn = pltpu.async_copy(nodes_hbm.at[pl.ds(mb, rows)],
                              idx2.at[buf, pl.ds(0, rows)], sem_ldn)
        dr = pltpu.async_copy(m_hbm.at[pl.ds(mb, rows)],
                              rows2.at[buf, pl.ds(0, rows)], sem_ldr)
        return dn, dr

    ld0 = fire_load(0, 0, HB)
    ld1 = fire_load(1, 1, HB)

    def _init(i, _):
        z16_v[i, :] = zvec
        return 0
    lax.fori_loop(0, RT, _init, 0)
    for i in range(CK):
        ones_v[i, :] = onevec
    for g in range(RT // L):
        riota_v[g * L // CK, pl.ds(g * L % CK, L)] = \
            lax.iota(jnp.int32, L) + (rbase + g * L)

    dz = pltpu.async_copy(zeros_hbm, sums_sh.at[pl.ds(rbase, RT)], sem_wr)
    dc = [pltpu.async_copy(z16_v.at[pl.ds(j * CK, CK)],
                           cnts_sh.at[riota_v.at[j]], sem_sco)
          for j in range(RT // CK)]
    dz.wait()
    for d in dc:
        d.wait()
    plsc.subcore_barrier()

    def remap(buf, ngroups):
        for g in range(ngroups):
            v = idx2[buf, pl.ds(g * L, L)]
            inr = (v >= nbase) & (v < nbase + NH)
            lidx2[buf, g * L // CK, pl.ds(g * L % CK, L)] = \
                jnp.where(inr, v - nbase, DUMMY)

    def fire_scatter(buf, nchunks):
        ds = []
        for j in range(nchunks):
            ds.append(pltpu.async_copy(rows2.at[buf, pl.ds(j * CK, CK)],
                                       sums_sh.at[lidx2.at[buf, j]],
                                       sem_scr, add=True))
            ds.append(pltpu.async_copy(ones_v,
                                       cnts_sh.at[lidx2.at[buf, j]],
                                       sem_sco, add=True))
        return ds

    def wait_all(ds):
        for d in ds:
            d.wait()

    ld0[0].wait(); ld0[1].wait()
    remap(0, HB // L)
    sc0 = fire_scatter(0, HB // CK)

    ld1[0].wait(); ld1[1].wait()
    remap(1, HB // L)
    sc1 = fire_scatter(1, HB // CK)

    @pl.when(sub < NS - 1)
    def _steps_23():
        wait_all(sc0)
        ld2 = fire_load(2, 0, HB)
        ld2[0].wait(); ld2[1].wait()
        remap(0, HB // L)
        sc2 = fire_scatter(0, HB // CK)
        wait_all(sc1)
        ld3 = fire_load(3, 1, HB)
        ld3[0].wait(); ld3[1].wait()
        remap(1, HB // L)
        sc3 = fire_scatter(1, HB // CK)
        wait_all(sc2)
        wait_all(sc3)

    @pl.when(sub == NS - 1)
    def _step_2t():
        wait_all(sc0)
        ldt = fire_load(2, 0, CK)
        ldt[0].wait(); ldt[1].wait()
        remap(0, CK // L)
        sct = fire_scatter(0, 1)
        wait_all(sct)
        wait_all(sc1)

    plsc.subcore_barrier()

    rd0 = pltpu.async_copy(sums_sh.at[pl.ds(rbase, HB)], rows2.at[0], sem_ldr)
    rd1 = pltpu.async_copy(sums_sh.at[pl.ds(rbase + HB, HB)], rows2.at[1],
                           sem_ldr)
    for j in range(RT // CK):
        pltpu.sync_copy(cnts_sh.at[riota_v.at[j]], z16_v.at[pl.ds(j * CK, CK)])

    def _scale(h, buf):
        def body(r, _):
            cvec = z16_v[h * HB + r, :]
            s = jnp.where(cvec > 0, 1.0 / cvec, 0.0)[0]
            for c in range(D // L):
                rows2[buf, r, pl.ds(c * L, L)] = \
                    rows2[buf, r, pl.ds(c * L, L)] * s
            return 0
        lax.fori_loop(0, HB, body, 0)

    rd0.wait()
    _scale(0, 0)
    wr0 = pltpu.async_copy(rows2.at[0], out_hbm.at[pl.ds(core * NH + rbase, HB)],
                           sem_wr)
    rd1.wait()
    _scale(1, 1)

    @pl.when(sub < NS - 1)
    def _write_full():
        pltpu.async_copy(rows2.at[1],
                         out_hbm.at[pl.ds(core * NH + rbase + HB, HB)],
                         sem_wr).wait()

    @pl.when(sub == NS - 1)
    def _write_short():
        last = NH - (NS - 1) * RT - HB
        pltpu.async_copy(rows2.at[1, pl.ds(0, last)],
                         out_hbm.at[pl.ds(core * NH + rbase + HB, last)],
                         sem_wr).wait()

    wr0.wait()


_agg = pl.kernel(
    _body,
    out_type=jax.ShapeDtypeStruct((N, D), jnp.float32),
    mesh=plsc.VectorSubcoreMesh(core_axis_name="c", subcore_axis_name="s",
                                num_cores=NC, num_subcores=NS),
    compiler_params=pltpu.CompilerParams(use_tc_tiling_on_sc=False),
    scratch_types=[
        pltpu.VMEM_SHARED((NHP, D), jnp.float32),
        pltpu.VMEM_SHARED((NHP, L), jnp.float32),
        pltpu.VMEM((2, HB, D), jnp.float32),
        pltpu.VMEM((2, HB), jnp.int32),
        pltpu.VMEM((2, 2, CK), jnp.int32),
        pltpu.VMEM((RT // CK, CK), jnp.int32),
        pltpu.VMEM((CK, L), jnp.float32),
        pltpu.VMEM((RT, L), jnp.float32),
        pltpu.SemaphoreType.DMA,
        pltpu.SemaphoreType.DMA,
        pltpu.SemaphoreType.DMA,
        pltpu.SemaphoreType.DMA,
        pltpu.SemaphoreType.DMA,
    ],
)


@jax.jit
def kernel(M, nodes):
    zeros = jnp.zeros((RT, D), jnp.float32)
    return _agg(M, nodes.astype(jnp.int32), zeros)


# speedup vs baseline: 4.3255x; 1.0024x over previous
import jax
import jax.numpy as jnp
from jax import lax
from jax.experimental import pallas as pl
from jax.experimental.pallas import tpu as pltpu
from jax.experimental.pallas import tpu_sc as plsc

N = 10000
D = 128
NUM_MSG = 10000
NC = 2
NS = 16
L = 16

HB = 160
CK = 80
MPT = 640
NH = N // NC
NHP = 5120
DUMMY = NH
RT = 320


def _body(m_hbm, nodes_hbm, zeros_hbm, out_hbm,
          sums_sh, cnts_sh,
          rows2, idx2, lidx2, riota_v, ones_v, z16_v,
          sem_ldn, sem_ldr, sem_scr, sem_sco, sem_wr):
    core = lax.axis_index("c")
    sub = lax.axis_index("s")
    zvec = jnp.zeros((L,), jnp.float32)
    onevec = jnp.ones((L,), jnp.float32)
    nbase = core * NH
    rbase = sub * RT
    mbase = sub * MPT

    def fire_load(step, buf, rows):
        mb = mbase + step * HB
        dn = pltpu.async_copy(nodes_hbm.at[pl.ds(mb, rows)],
                              idx2.at[buf, pl.ds(0, rows)], sem_ldn)
        dr = pltpu.async_copy(m_hbm.at[pl.ds(mb, rows)],
                              rows2.at[buf, pl.ds(0, rows)], sem_ldr)
        return dn, dr

    scope = jax.named_scope
    with scope("ph_prime"):
        ld0 = fire_load(0, 0, HB)
    ld1 = fire_load(1, 1, HB)

    def _init(i, _):
        z16_v[i, :] = zvec
        return 0
    lax.fori_loop(0, RT, _init, 0)
    for i in range(CK):
        ones_v[i, :] = onevec
    for g in range(RT // L):
        riota_v[g * L // CK, pl.ds(g * L % CK, L)] = \
            lax.iota(jnp.int32, L) + (rbase + g * L)

    dz = pltpu.async_copy(zeros_hbm, sums_sh.at[pl.ds(rbase, RT)], sem_wr)
    dc = [pltpu.async_copy(z16_v.at[pl.ds(j * CK, CK)],
                           cnts_sh.at[riota_v.at[j]], sem_sco)
          for j in range(RT // CK)]
    dz.wait()
    for d in dc:
        d.wait()
    plsc.subcore_barrier()

    def remap(buf, ngroups):
        for g in range(ngroups):
            v = idx2[buf, pl.ds(g * L, L)]
            inr = (v >= nbase) & (v < nbase + NH)
            lidx2[buf, g * L // CK, pl.ds(g * L % CK, L)] = \
                jnp.where(inr, v - nbase, DUMMY)

    def fire_scatter(buf, nchunks):
        ds = []
        for j in range(nchunks):
            ds.append(pltpu.async_copy(rows2.at[buf, pl.ds(j * CK, CK)],
                                       sums_sh.at[lidx2.at[buf, j]],
                                       sem_scr, add=True))
            ds.append(pltpu.async_copy(ones_v,
                                       cnts_sh.at[lidx2.at[buf, j]],
                                       sem_sco, add=True))
        return ds

    def wait_all(ds):
        for d in ds:
            d.wait()

    ld0[0].wait(); ld0[1].wait()
    remap(0, HB // L)
    sc0 = fire_scatter(0, HB // CK)

    ld1[0].wait(); ld1[1].wait()
    remap(1, HB // L)
    sc1 = fire_scatter(1, HB // CK)

    @pl.when(sub < NS - 1)
    def _steps_23():
        wait_all(sc0)
        ld2 = fire_load(2, 0, HB)
        ld2[0].wait(); ld2[1].wait()
        remap(0, HB // L)
        sc2 = fire_scatter(0, HB // CK)
        wait_all(sc1)
        ld3 = fire_load(3, 1, HB)
        ld3[0].wait(); ld3[1].wait()
        remap(1, HB // L)
        sc3 = fire_scatter(1, HB // CK)
        wait_all(sc2)
        wait_all(sc3)

    @pl.when(sub == NS - 1)
    def _step_2t():
        wait_all(sc0)
        ldt = fire_load(2, 0, CK)
        ldt[0].wait(); ldt[1].wait()
        remap(0, CK // L)
        sct = fire_scatter(0, 1)
        wait_all(sct)
        wait_all(sc1)

    plsc.subcore_barrier()

    rd0 = pltpu.async_copy(sums_sh.at[pl.ds(rbase, HB)], rows2.at[0], sem_ldr)
    rd1 = pltpu.async_copy(sums_sh.at[pl.ds(rbase + HB, HB)], rows2.at[1],
                           sem_ldr)
    for j in range(RT // CK):
        pltpu.sync_copy(cnts_sh.at[riota_v.at[j]], z16_v.at[pl.ds(j * CK, CK)])

    def _scale(h, buf):
        def body(r, _):
            cvec = z16_v[h * HB + r, :]
            s = jnp.where(cvec > 0, 1.0 / cvec, 0.0)[0]
            for c in range(D // L):
                rows2[buf, r, pl.ds(c * L, L)] = \
                    rows2[buf, r, pl.ds(c * L, L)] * s
            return 0
        lax.fori_loop(0, HB, body, 0)

    rd0.wait()
    _scale(0, 0)
    wr0 = pltpu.async_copy(rows2.at[0], out_hbm.at[pl.ds(core * NH + rbase, HB)],
                           sem_wr)
    rd1.wait()
    _scale(1, 1)

    @pl.when(sub < NS - 1)
    def _write_full():
        pltpu.async_copy(rows2.at[1],
                         out_hbm.at[pl.ds(core * NH + rbase + HB, HB)],
                         sem_wr).wait()

    @pl.when(sub == NS - 1)
    def _write_short():
        last = NH - (NS - 1) * RT - HB
        pltpu.async_copy(rows2.at[1, pl.ds(0, last)],
                         out_hbm.at[pl.ds(core * NH + rbase + HB, last)],
                         sem_wr).wait()

    wr0.wait()


_agg = pl.kernel(
    _body,
    out_type=jax.ShapeDtypeStruct((N, D), jnp.float32),
    mesh=plsc.VectorSubcoreMesh(core_axis_name="c", subcore_axis_name="s",
                                num_cores=NC, num_subcores=NS),
    compiler_params=pltpu.CompilerParams(use_tc_tiling_on_sc=False),
    scratch_types=[
        pltpu.VMEM_SHARED((NHP, D), jnp.float32),
        pltpu.VMEM_SHARED((NHP, L), jnp.float32),
        pltpu.VMEM((2, HB, D), jnp.float32),
        pltpu.VMEM((2, HB), jnp.int32),
        pltpu.VMEM((2, 2, CK), jnp.int32),
        pltpu.VMEM((RT // CK, CK), jnp.int32),
        pltpu.VMEM((CK, L), jnp.float32),
        pltpu.VMEM((RT, L), jnp.float32),
        pltpu.SemaphoreType.DMA,
        pltpu.SemaphoreType.DMA,
        pltpu.SemaphoreType.DMA,
        pltpu.SemaphoreType.DMA,
        pltpu.SemaphoreType.DMA,
    ],
)


@jax.jit
def kernel(M, nodes):
    zeros = jnp.zeros((RT, D), jnp.float32)
    return _agg(M, nodes.astype(jnp.int32), zeros)


# speedup vs baseline: 4.4212x; 1.0221x over previous
import jax
import jax.numpy as jnp
from jax import lax
from jax.experimental import pallas as pl
from jax.experimental.pallas import tpu as pltpu
from jax.experimental.pallas import tpu_sc as plsc

N = 10000
D = 128
DH = 64
NUM_MSG = 10000
NC = 2
NS = 16
L = 16

HB = 160
CK = 80
MPT = 640
NHP = 10240
RT = 640


def _body(m_hbm, nodes2_hbm, zeros_hbm, out_hbm,
          sums_sh, cnts_sh,
          rows2, lidx2, ones_v, z16_v,
          sem_ldn, sem_ldr, sem_scr, sem_sco, sem_wr, sem_rd):
    core = lax.axis_index("c")
    sub = lax.axis_index("s")
    onevec = jnp.ones((L,), jnp.float32)
    cb = core * DH
    rbase = sub * RT
    mbase = sub * MPT
    bbase = sub * (MPT // CK)

    def fire_load(step, buf, rows):
        mb = mbase + step * HB
        blk = bbase + step * (HB // CK)
        dn = pltpu.async_copy(nodes2_hbm.at[pl.ds(blk, rows // CK)],
                              lidx2.at[buf, pl.ds(0, rows // CK)], sem_ldn)
        dr = pltpu.async_copy(m_hbm.at[pl.ds(mb, rows), pl.ds(cb, DH)],
                              rows2.at[buf, pl.ds(0, rows)], sem_ldr)
        return dn, dr

    ld0 = fire_load(0, 0, HB)
    ld1 = fire_load(1, 1, HB)

    for i in range(CK):
        ones_v[i, :] = onevec
    dz = pltpu.async_copy(zeros_hbm, sums_sh.at[pl.ds(rbase, RT)], sem_wr)
    dc = pltpu.async_copy(zeros_hbm.at[:, pl.ds(0, L)],
                          cnts_sh.at[pl.ds(rbase, RT)], sem_wr)
    dz.wait()
    dc.wait()
    plsc.subcore_barrier()

    def fire_scatter(buf, nchunks):
        ds = []
        for j in range(nchunks):
            ds.append(pltpu.async_copy(rows2.at[buf, pl.ds(j * CK, CK)],
                                       sums_sh.at[lidx2.at[buf, j]],
                                       sem_scr, add=True))
            ds.append(pltpu.async_copy(ones_v,
                                       cnts_sh.at[lidx2.at[buf, j]],
                                       sem_sco, add=True))
        return ds

    def wait_all(ds):
        for d in ds:
            d.wait()

    ld0[0].wait(); ld0[1].wait()
    sc0 = fire_scatter(0, HB // CK)

    ld1[0].wait(); ld1[1].wait()
    sc1 = fire_scatter(1, HB // CK)

    @pl.when(sub < NS - 1)
    def _steps_23():
        wait_all(sc0)
        ld2 = fire_load(2, 0, HB)
        ld2[0].wait(); ld2[1].wait()
        sc2 = fire_scatter(0, HB // CK)
        wait_all(sc1)
        ld3 = fire_load(3, 1, HB)
        ld3[0].wait(); ld3[1].wait()
        sc3 = fire_scatter(1, HB // CK)
        wait_all(sc2)
        wait_all(sc3)

    @pl.when(sub == NS - 1)
    def _step_2t():
        wait_all(sc0)
        ldt = fire_load(2, 0, CK)
        ldt[0].wait(); ldt[1].wait()
        sct = fire_scatter(0, 1)
        wait_all(sct)
        wait_all(sc1)

    plsc.subcore_barrier()

    dcr = pltpu.async_copy(cnts_sh.at[pl.ds(rbase, RT)], z16_v, sem_rd)

    def read_half(h, buf, rows):
        return pltpu.async_copy(sums_sh.at[pl.ds(rbase + h * HB, rows)],
                                rows2.at[buf, pl.ds(0, rows)], sem_ldr)

    def scale_half(h, buf, rows):
        def body(r, _):
            cvec = z16_v[h * HB + r, :]
            s = jnp.where(cvec > 0, 1.0 / cvec, 0.0)[0]
            for c in range(DH // L):
                rows2[buf, r, pl.ds(c * L, L)] = \
                    rows2[buf, r, pl.ds(c * L, L)] * s
            return 0
        lax.fori_loop(0, rows, body, 0)

    def write_half(h, buf, rows):
        return pltpu.async_copy(
            rows2.at[buf, pl.ds(0, rows)],
            out_hbm.at[pl.ds(rbase + h * HB, rows), pl.ds(cb, DH)], sem_wr)

    rd0 = read_half(0, 0, HB)
    rd1 = read_half(1, 1, HB)
    dcr.wait()
    rd0.wait()
    scale_half(0, 0, HB)
    wr0 = write_half(0, 0, HB)
    rd1.wait()
    scale_half(1, 1, HB)
    wr1 = write_half(1, 1, HB)

    @pl.when(sub < NS - 1)
    def _out_full():
        wr0.wait()
        rd2 = read_half(2, 0, HB)
        rd2.wait()
        scale_half(2, 0, HB)
        wr2 = write_half(2, 0, HB)
        wr1.wait()
        rd3 = read_half(3, 1, HB)
        rd3.wait()
        scale_half(3, 1, HB)
        wr3 = write_half(3, 1, HB)
        wr2.wait()
        wr3.wait()

    @pl.when(sub == NS - 1)
    def _out_short():
        wr0.wait()
        rdt = read_half(2, 0, CK)
        rdt.wait()
        scale_half(2, 0, CK)
        wrt = write_half(2, 0, CK)
        wrt.wait()
        wr1.wait()


_agg = pl.kernel(
    _body,
    out_type=jax.ShapeDtypeStruct((N, D), jnp.float32),
    mesh=plsc.VectorSubcoreMesh(core_axis_name="c", subcore_axis_name="s",
                                num_cores=NC, num_subcores=NS),
    compiler_params=pltpu.CompilerParams(use_tc_tiling_on_sc=False),
    scratch_types=[
        pltpu.VMEM_SHARED((NHP, DH), jnp.float32),
        pltpu.VMEM_SHARED((NHP, L), jnp.float32),
        pltpu.VMEM((2, HB, DH), jnp.float32),
        pltpu.VMEM((2, HB // CK, CK), jnp.int32),
        pltpu.VMEM((CK, L), jnp.float32),
        pltpu.VMEM((RT, L), jnp.float32),
        pltpu.SemaphoreType.DMA,
        pltpu.SemaphoreType.DMA,
        pltpu.SemaphoreType.DMA,
        pltpu.SemaphoreType.DMA,
        pltpu.SemaphoreType.DMA,
        pltpu.SemaphoreType.DMA,
    ],
)


@jax.jit
def kernel(M, nodes):
    zeros = jnp.zeros((RT, DH), jnp.float32)
    nodes2 = nodes.astype(jnp.int32).reshape(NUM_MSG // CK, CK)
    return _agg(M, nodes2, zeros)
